# baseline (speedup 1.0000x reference)
import functools

import jax
import jax.numpy as jnp
from jax import lax
from jax.experimental import pallas as pl
from jax.experimental.pallas import tpu as pltpu
from jax.experimental.pallas import tpu_sc as plsc

_INFO = plsc.get_sparse_core_info()
_NC, _NS, _L = _INFO.num_cores, _INFO.num_subcores, _INFO.num_lanes
_NW = _NC * _NS

_NB = 8
_MAGIC = 1.5 * 2.0 ** 23


@functools.lru_cache(maxsize=None)
def _make_quantize(nrow: int, ncol: int, k: int):
    rows_w = nrow // _NW
    rch = rows_w // _NB
    assert nrow % _NW == 0 and rows_w % _NB == 0 and ncol % _L == 0
    assert k >= _L
    n_vec = ncol // _L
    mesh = plsc.VectorSubcoreMesh(core_axis_name="c", subcore_axis_name="s")

    @functools.partial(
        pl.kernel,
        mesh=mesh,
        out_type=jax.ShapeDtypeStruct((nrow, ncol), jnp.float32),
        scratch_types=[pltpu.VMEM((rch, ncol), jnp.float32)] * _NB + [
            pltpu.VMEM((4 * _L,), jnp.float32),
            pltpu.VMEM((k,), jnp.float32),
        ] + [pltpu.SemaphoreType.DMA] * (2 * _NB),
    )
    def _quantize(x_hbm, cb_hbm, out_hbm, *refs):
        bufs = refs[:_NB]
        consts_v = refs[_NB]
        cb_v = refs[_NB + 1]
        isems = refs[_NB + 2:2 * _NB + 2]
        osems = refs[2 * _NB + 2:]
        wid = lax.axis_index("s") * _NC + lax.axis_index("c")
        row0 = wid * rows_w

        hin = [
            pltpu.async_copy(
                x_hbm.at[pl.ds(row0 + j * rch, rch)], bufs[j], isems[j])
            for j in range(_NB)
        ]

        pltpu.sync_copy(cb_hbm, cb_v)

        def bcast_lane(vec, lane):
            idx = jnp.full((_L, 1), lane, jnp.int32)
            dn = lax.GatherDimensionNumbers(
                offset_dims=(), collapsed_slice_dims=(0,),
                start_index_map=(0,))
            return lax.gather(vec, idx, dn, (1,),
                              mode=lax.GatherScatterMode.PROMISE_IN_BOUNDS)

        c0_g = bcast_lane(cb_v[pl.ds(0, _L)], 0)
        c_last = bcast_lane(cb_v[pl.ds(k - _L, _L)], _L - 1)
        step_g = (c_last - c0_g) * jnp.float32(1.0 / (k - 1))
        inv_g = jnp.float32(1.0) / step_g
        consts_v[pl.ds(0, _L)] = -c0_g * inv_g
        consts_v[pl.ds(_L, _L)] = c0_g
        consts_v[pl.ds(2 * _L, _L)] = step_g
        consts_v[pl.ds(3 * _L, _L)] = inv_g
        bias = consts_v[pl.ds(0, _L)]
        c0 = consts_v[pl.ds(_L, _L)]
        step = consts_v[pl.ds(2 * _L, _L)]
        inv = consts_v[pl.ds(3 * _L, _L)]
        ubound = jnp.full((_L,), float(k - 1), jnp.float32)
        zero = jnp.zeros((_L,), jnp.float32)
        magic = jnp.full((_L,), _MAGIC, jnp.float32)

        hout = []
        for j in range(_NB):
            hin[j].wait()
            buf = bufs[j]

            @plsc.parallel_loop(0, n_vec, unroll=2)
            def _loop(i, buf=buf):
                for r in range(rch):
                    x = buf[r, pl.ds(i * _L, _L)]
                    t = x * inv + bias
                    t = jnp.minimum(jnp.maximum(t, zero), ubound)
                    idx_f = (t + magic) - magic
                    buf[r, pl.ds(i * _L, _L)] = c0 + idx_f * step
            hout.append(pltpu.async_copy(
                buf, out_hbm.at[pl.ds(row0 + j * rch, rch)], osems[j]))
        for h in hout:
            h.wait()

    return _quantize


def kernel(input, codebook):
    nrow, ncol = input.shape
    k = codebook.shape[0]
    return _make_quantize(nrow, ncol, k)(input, codebook)


# speedup vs baseline: 1.0608x; 1.0608x over previous
import functools

import jax
import jax.numpy as jnp
from jax import lax
from jax.experimental import pallas as pl
from jax.experimental.pallas import tpu as pltpu
from jax.experimental.pallas import tpu_sc as plsc

_INFO = plsc.get_sparse_core_info()
_NC, _NS, _L = _INFO.num_cores, _INFO.num_subcores, _INFO.num_lanes
_NW = _NC * _NS

_NB = 8
_MAGIC = 1.5 * 2.0 ** 23


@functools.lru_cache(maxsize=None)
def _make_quantize(nrow: int, ncol: int, k: int):
    rows_w = nrow // _NW
    rch = rows_w // _NB
    assert nrow % _NW == 0 and rows_w % _NB == 0 and ncol % _L == 0
    assert k >= _L
    n_vec = ncol // _L
    mesh = plsc.VectorSubcoreMesh(core_axis_name="c", subcore_axis_name="s")

    @functools.partial(
        pl.kernel,
        mesh=mesh,
        out_type=jax.ShapeDtypeStruct((nrow, ncol), jnp.float32),
        scratch_types=[pltpu.VMEM((rch, ncol), jnp.float32)] * _NB + [
            pltpu.VMEM((4 * _L,), jnp.float32),
            pltpu.VMEM((k,), jnp.float32),
        ] + [pltpu.SemaphoreType.DMA] * (2 * _NB),
    )
    def _quantize(x_hbm, cb_hbm, out_hbm, *refs):
        bufs = refs[:_NB]
        consts_v = refs[_NB]
        cb_v = refs[_NB + 1]
        isems = refs[_NB + 2:2 * _NB + 2]
        osems = refs[2 * _NB + 2:]
        wid = lax.axis_index("s") * _NC + lax.axis_index("c")
        row0 = wid * rows_w

        hin = [
            pltpu.async_copy(
                x_hbm.at[pl.ds(row0 + j * rch, rch)], bufs[j], isems[j])
            for j in range(_NB)
        ]

        pltpu.sync_copy(cb_hbm, cb_v)

        def bcast_lane(vec, lane):
            idx = jnp.full((_L, 1), lane, jnp.int32)
            dn = lax.GatherDimensionNumbers(
                offset_dims=(), collapsed_slice_dims=(0,),
                start_index_map=(0,))
            return lax.gather(vec, idx, dn, (1,),
                              mode=lax.GatherScatterMode.PROMISE_IN_BOUNDS)

        c0_g = bcast_lane(cb_v[pl.ds(0, _L)], 0)
        c_last = bcast_lane(cb_v[pl.ds(k - _L, _L)], _L - 1)
        step_g = (c_last - c0_g) * jnp.float32(1.0 / (k - 1))
        inv_g = jnp.float32(1.0) / step_g
        consts_v[pl.ds(0, _L)] = -c0_g * inv_g
        consts_v[pl.ds(_L, _L)] = c0_g
        consts_v[pl.ds(2 * _L, _L)] = step_g
        consts_v[pl.ds(3 * _L, _L)] = inv_g
        bias = consts_v[pl.ds(0, _L)]
        c0 = consts_v[pl.ds(_L, _L)]
        step = consts_v[pl.ds(2 * _L, _L)]
        inv = consts_v[pl.ds(3 * _L, _L)]
        ubound = jnp.full((_L,), float(k - 1), jnp.float32)
        zero = jnp.zeros((_L,), jnp.float32)
        magic = jnp.full((_L,), _MAGIC, jnp.float32)

        hout = []
        for j in range(_NB):
            hin[j].wait()
            buf = bufs[j]

            @plsc.parallel_loop(0, n_vec)
            def _loop(i, buf=buf):
                for r in range(rch):
                    x = buf[r, pl.ds(i * _L, _L)]
                    t = x * inv + bias
                    t = jnp.minimum(jnp.maximum(t, zero), ubound)
                    idx_f = (t + magic) - magic
                    buf[r, pl.ds(i * _L, _L)] = c0 + idx_f * step
            hout.append(pltpu.async_copy(
                buf, out_hbm.at[pl.ds(row0 + j * rch, rch)], osems[j]))
        for h in hout:
            h.wait()

    return _quantize


def kernel(input, codebook):
    nrow, ncol = input.shape
    k = codebook.shape[0]
    return _make_quantize(nrow, ncol, k)(input, codebook)
